# baseline (speedup 1.0000x reference)
import functools
import jax
import jax.numpy as jnp
import numpy as np
from jax.experimental import pallas as pl
from jax.experimental.pallas import tpu as pltpu

PROB = 0.5
TWO_PI = float(2.0 * np.pi)
BLK = 4096

def _tc_launch(B, F, P):
    def body(x_ref, rot_ref, bool_ref, cA, cB, cC, cD, cE, cF, sel_ref, o_ref):
        x = x_ref[...]
        rot = rot_ref[...] * TWO_PI
        mask = bool_ref[...] < PROB
        s = x * cA[...] + cB[...] + rot
        r = jnp.remainder(s, TWO_PI)
        v_rot = r * cC[...] + cD[...]
        v_keep = x * cE[...] + cF[...]
        new = jnp.where(mask, v_rot, v_keep)
        o_ref[...] = jnp.where(sel_ref[...] > 0, new, x)

    grid = (B // BLK,)
    return pl.pallas_call(
        body,
        grid=grid,
        in_specs=[
            pl.BlockSpec((BLK, F), lambda i: (i, 0)),
            pl.BlockSpec((BLK, 1), lambda i: (i, 0)),
            pl.BlockSpec((BLK, 1), lambda i: (i, 0)),
        ] + [pl.BlockSpec((1, F), lambda i: (0, 0))] * 7,
        out_specs=pl.BlockSpec((BLK, F), lambda i: (i, 0)),
        out_shape=jax.ShapeDtypeStruct((B, F), jnp.float32),
    )

def kernel(x, bool_rand, rot_rand, l1_scale, scale, bias, phi_indices):
    B, F = x.shape
    P = phi_indices.shape[0]
    inv_l1 = 1.0 / l1_scale
    inv_s = 1.0 / scale
    A = scale * inv_l1
    Bc = bias * inv_l1
    C = l1_scale * inv_s
    D = -bias * inv_s
    E = inv_l1
    Fc = (Bc - bias) * inv_s
    def expand(v, fill):
        t = jnp.full((F,), fill, jnp.float32)
        return t.at[phi_indices].set(v.astype(jnp.float32))[None, :]
    cA, cB_, cC, cD_, cE, cF_ = (expand(A,1.), expand(Bc,0.), expand(C,1.),
                                 expand(D,0.), expand(E,1.), expand(Fc,0.))
    sel = jnp.zeros((F,), jnp.int32).at[phi_indices].set(1)[None, :]
    launch = _tc_launch(B, F, P)
    return launch(x, rot_rand.astype(jnp.float32)[:, None],
                  bool_rand.astype(jnp.float32)[:, None],
                  cA, cB_, cC, cD_, cE, cF_, sel)
